# baseline (speedup 1.0000x reference)
import jax
import jax.numpy as jnp
from jax import lax
from jax.experimental import pallas as pl
from jax.experimental.pallas import tpu as pltpu
from jax.experimental.pallas import tpu_sc as plsc

_N = 10000
_E = 320000
_D = 128
_G = 64
_STEPS = 3

_NC = 2
_NS = 16
_NW = _NC * _NS
_CH = 128
_NCHUNK = 80
_HC = _NCHUNK // 2
_EPAD = _NW * _NCHUNK * _CH
_NPAD = 10240
_STRIPE = _NPAD // _NS


def _deg_body(dst_hbm, zerov_hbm, out_hbm, idx_d, ones, accd):
    cid = lax.axis_index("c")
    sid = lax.axis_index("s")
    wid = cid * _NS + sid
    pltpu.sync_copy(dst_hbm.at[wid], idx_d)
    pltpu.sync_copy(zerov_hbm.at[pl.ds(sid * _STRIPE, _STRIPE)],
                    accd.at[pl.ds(sid * _STRIPE, _STRIPE)])
    for k in range(_CH // 16):
        ones[pl.ds(k * 16, 16)] = jnp.ones((16,), jnp.float32)
    plsc.subcore_barrier()

    def step(j, carry):
        pltpu.sync_copy(ones, accd.at[idx_d.at[j]], add=True)
        return carry

    lax.fori_loop(0, _NCHUNK, step, 0)
    plsc.subcore_barrier()
    pltpu.sync_copy(accd.at[pl.ds(sid * _STRIPE, _STRIPE)],
                    out_hbm.at[cid, pl.ds(sid * _STRIPE, _STRIPE)])


import functools


@functools.lru_cache(maxsize=None)
def _sc_mesh():
    return plsc.VectorSubcoreMesh(core_axis_name="c", subcore_axis_name="s",
                                  num_cores=_NC, num_subcores=_NS)


@functools.lru_cache(maxsize=None)
def _deg_kernel_build():
    return pl.kernel(
        _deg_body,
        out_type=jax.ShapeDtypeStruct((_NC, _NPAD), jnp.float32),
        mesh=_sc_mesh(),
        scratch_types=[
            pltpu.VMEM((_NCHUNK, _CH), jnp.int32),
            pltpu.VMEM((_CH,), jnp.float32),
            pltpu.VMEM_SHARED((_NPAD,), jnp.float32),
        ],
    )


def _edge_scatter_body(xs_hbm, src_hbm, dst_hbm, zrow_hbm, out_hbm,
                       idx_s, idx_d, rows0, rows1, acc, sem0, sem1, sem2,
                       sem3):
    cid = lax.axis_index("c")
    sid = lax.axis_index("s")
    wid = cid * _NS + sid
    pltpu.sync_copy(zrow_hbm.at[pl.ds(sid * _STRIPE, _STRIPE)],
                    acc.at[pl.ds(sid * _STRIPE, _STRIPE)])
    plsc.subcore_barrier()

    def gather_start(j, buf, sem):
        pltpu.async_copy(xs_hbm.at[idx_s.at[j]], buf, sem)

    def gather_wait(j, buf, sem):
        pltpu.make_async_copy(xs_hbm.at[idx_s.at[j]], buf, sem).wait()

    def scat_start(j, buf, sem):
        pltpu.async_copy(buf, acc.at[idx_d.at[j]], sem, add=True)

    def scat_wait(j, buf, sem):
        pltpu.make_async_copy(buf, acc.at[idx_d.at[j]], sem).wait()

    for h in range(_NCHUNK // _HC):
        pltpu.sync_copy(src_hbm.at[wid, pl.ds(h * _HC, _HC)], idx_s)
        pltpu.sync_copy(dst_hbm.at[wid, pl.ds(h * _HC, _HC)], idx_d)
        gather_start(0, rows0, sem0)
        gather_start(1, rows1, sem1)

        def step(i, carry):
            j = 2 * i
            gather_wait(j, rows0, sem0)
            scat_start(j, rows0, sem2)
            gather_wait(j + 1, rows1, sem1)
            scat_start(j + 1, rows1, sem3)
            scat_wait(j, rows0, sem2)

            @pl.when(i + 1 < _HC // 2)
            def _():
                gather_start(j + 2, rows0, sem0)

            scat_wait(j + 1, rows1, sem3)

            @pl.when(i + 1 < _HC // 2)
            def _():
                gather_start(j + 3, rows1, sem1)

            return carry

        lax.fori_loop(0, _HC // 2, step, 0)
    plsc.subcore_barrier()
    pltpu.sync_copy(acc.at[pl.ds(sid * _STRIPE, _STRIPE)],
                    out_hbm.at[cid, pl.ds(sid * _STRIPE, _STRIPE)])


@functools.lru_cache(maxsize=None)
def _edge_scatter_build():
    return pl.kernel(
        _edge_scatter_body,
        out_type=jax.ShapeDtypeStruct((_NC, _NPAD, _D), jnp.float32),
        mesh=_sc_mesh(),
        scratch_types=[
            pltpu.VMEM((_HC, _CH), jnp.int32),
            pltpu.VMEM((_HC, _CH), jnp.int32),
            pltpu.VMEM((_CH, _D), jnp.float32),
            pltpu.VMEM((_CH, _D), jnp.float32),
            pltpu.VMEM_SHARED((_NPAD, _D), jnp.float32),
            pltpu.SemaphoreType.DMA,
            pltpu.SemaphoreType.DMA,
            pltpu.SemaphoreType.DMA,
            pltpu.SemaphoreType.DMA,
        ],
    )


def _dinv_col(degp):
    deg = degp[0, :] + degp[1, :] + 1.0
    return lax.rsqrt(deg).reshape(_NPAD, 1)


def _sigmoid(v):
    return 1.0 / (1.0 + jnp.exp(-v))


def _tc_a_body(x_ref, w_ref, degp_ref, xw_ref, xs_ref):
    dc = _dinv_col(degp_ref[...])
    xw = jnp.dot(x_ref[...], w_ref[...], preferred_element_type=jnp.float32)
    xw_ref[...] = xw
    xs_ref[...] = xw * dc


def _tc_b_body(acc_ref, xw1_ref, w2_ref, b1_ref, degp_ref, xw2_ref, xs2_ref):
    dc = _dinv_col(degp_ref[...])
    agg = acc_ref[0] + acc_ref[1]
    h1 = jnp.maximum(dc * agg + dc * dc * xw1_ref[...] + b1_ref[...][None, :],
                     0.0)
    xw2 = jnp.dot(h1, w2_ref[...], preferred_element_type=jnp.float32)
    xw2_ref[...] = xw2
    xs2_ref[...] = xw2 * dc


def _tc_c_body(acc_ref, xw2_ref, b2_ref, degp_ref, batch_ref,
               wi_ref, wh_ref, bi_ref, bh_ref, out_ref):
    f32 = jnp.float32
    dc = _dinv_col(degp_ref[...])
    h2 = (dc * (acc_ref[0] + acc_ref[1]) + dc * dc * xw2_ref[...]
          + b2_ref[...][None, :])
    bat = batch_ref[...]
    gids = lax.broadcasted_iota(jnp.int32, (_G, _NPAD), 0)
    seg = gids == bat[None, :]
    segf = seg.astype(f32)

    h = jnp.zeros((_G, _D), f32)
    c = jnp.zeros((_G, _D), f32)
    qs = jnp.zeros((_G, 2 * _D), f32)
    for _ in range(_STEPS):
        gates = (jnp.dot(qs, wi_ref[...], preferred_element_type=f32)
                 + jnp.dot(h, wh_ref[...], preferred_element_type=f32)
                 + bi_ref[...][None, :] + bh_ref[...][None, :])
        ii = _sigmoid(gates[:, 0:_D])
        ff = _sigmoid(gates[:, _D:2 * _D])
        gg = jnp.tanh(gates[:, 2 * _D:3 * _D])
        oo = _sigmoid(gates[:, 3 * _D:4 * _D])
        c = ff * c + ii * gg
        h = oo * jnp.tanh(c)
        q = h
        qh = lax.dot_general(q, h2, (((1,), (1,)), ((), ())),
                             preferred_element_type=f32)
        e = jnp.sum(jnp.where(seg, qh, 0.0), axis=0)
        m = jnp.max(jnp.where(seg, e[None, :], -jnp.inf), axis=1)
        m = jnp.where(jnp.abs(m) < jnp.inf, m, 0.0)
        mrow = jnp.sum(segf * m[:, None], axis=0)
        ex = jnp.exp(e - mrow)
        ext = segf * ex[None, :]
        ssum = jnp.sum(ext, axis=1)
        rnum = jnp.dot(ext, h2, preferred_element_type=f32)
        r = rnum / (ssum[:, None] + 1e-16)
        qs = jnp.concatenate([q, r], axis=1)
    out_ref[...] = qs


_tc_a = pl.pallas_call(
    _tc_a_body,
    out_shape=[jax.ShapeDtypeStruct((_NPAD, _D), jnp.float32)] * 2,
)

_tc_b = pl.pallas_call(
    _tc_b_body,
    out_shape=[jax.ShapeDtypeStruct((_NPAD, _D), jnp.float32)] * 2,
)

_tc_c = pl.pallas_call(
    _tc_c_body,
    out_shape=jax.ShapeDtypeStruct((_G, 2 * _D), jnp.float32),
)


def kernel(x, edge_index, batch, W1, b1, W2, b2, Wi, Wh, bi, bh):
    f32 = jnp.float32
    src = edge_index[0]
    dst = edge_index[1]
    pad_e = _EPAD - _E
    fill = _N + (jnp.arange(pad_e, dtype=jnp.int32) % (_NPAD - _N))
    srcp = jnp.concatenate([src, fill]).reshape(_NW, _NCHUNK, _CH)
    dstp = jnp.concatenate([dst, fill]).reshape(_NW, _NCHUNK, _CH)
    xp = jnp.concatenate([x, jnp.zeros((_NPAD - _N, _D), f32)], axis=0)
    batp = jnp.concatenate([batch, jnp.full((_NPAD - _N,), _G, jnp.int32)])
    zrow = jnp.zeros((_NPAD, _D), f32)
    zvec = jnp.zeros((_NPAD,), f32)

    degp = _deg_kernel_build()(dstp, zvec)
    xw1, xs1 = _tc_a(xp, W1, degp)
    acc1 = _edge_scatter_build()(xs1, srcp, dstp, zrow)
    xw2, xs2 = _tc_b(acc1, xw1, W2, b1, degp)
    acc2 = _edge_scatter_build()(xs2, srcp, dstp, zrow)
    return _tc_c(acc2, xw2, b2, degp, batp, Wi, Wh, bi, bh)


# speedup vs baseline: 1.2689x; 1.2689x over previous
import jax
import jax.numpy as jnp
from jax import lax
from jax.experimental import pallas as pl
from jax.experimental.pallas import tpu as pltpu
from jax.experimental.pallas import tpu_sc as plsc

_N = 10000
_E = 320000
_D = 128
_G = 64
_STEPS = 3

_NC = 2
_NS = 16
_NW = _NC * _NS
_CH = 128
_NCHUNK = 80
_HC = _NCHUNK // 2
_EPAD = _NW * _NCHUNK * _CH
_NPAD = 10240
_STRIPE = _NPAD // _NS


def _deg_body(dst_hbm, zerov_hbm, out_hbm, idx_d, ones, accd):
    cid = lax.axis_index("c")
    sid = lax.axis_index("s")
    wid = cid * _NS + sid
    pltpu.sync_copy(dst_hbm.at[wid], idx_d)
    pltpu.sync_copy(zerov_hbm.at[pl.ds(sid * _STRIPE, _STRIPE)],
                    accd.at[pl.ds(sid * _STRIPE, _STRIPE)])
    for k in range(_CH // 16):
        ones[pl.ds(k * 16, 16)] = jnp.ones((16,), jnp.float32)
    plsc.subcore_barrier()

    def step(j, carry):
        pltpu.sync_copy(ones, accd.at[idx_d.at[j]], add=True)
        return carry

    lax.fori_loop(0, _NCHUNK, step, 0)
    plsc.subcore_barrier()
    pltpu.sync_copy(accd.at[pl.ds(sid * _STRIPE, _STRIPE)],
                    out_hbm.at[cid, pl.ds(sid * _STRIPE, _STRIPE)])


import functools


@functools.lru_cache(maxsize=None)
def _sc_mesh():
    return plsc.VectorSubcoreMesh(core_axis_name="c", subcore_axis_name="s",
                                  num_cores=_NC, num_subcores=_NS)


@functools.lru_cache(maxsize=None)
def _deg_kernel_build():
    return pl.kernel(
        _deg_body,
        out_type=jax.ShapeDtypeStruct((_NC, _NPAD), jnp.float32),
        mesh=_sc_mesh(),
        scratch_types=[
            pltpu.VMEM((_NCHUNK, _CH), jnp.int32),
            pltpu.VMEM((_CH,), jnp.float32),
            pltpu.VMEM_SHARED((_NPAD,), jnp.float32),
        ],
    )


def _edge_scatter_body(xs_hbm, src_hbm, dst_hbm, zrow_hbm, out_hbm,
                       idx_s, idx_d, rows0, rows1, acc, sem0, sem1, sem2):
    cid = lax.axis_index("c")
    sid = lax.axis_index("s")
    wid = cid * _NS + sid
    zcp = pltpu.async_copy(zrow_hbm.at[pl.ds(sid * _STRIPE, _STRIPE)],
                           acc.at[pl.ds(sid * _STRIPE, _STRIPE)], sem2)

    def gather_start(j, buf, sem):
        pltpu.async_copy(xs_hbm.at[idx_s.at[j]], buf, sem)

    def gather_wait(j, buf, sem):
        pltpu.make_async_copy(xs_hbm.at[idx_s.at[j]], buf, sem).wait()

    first = True
    for h in range(_NCHUNK // _HC):
        pltpu.sync_copy(src_hbm.at[wid, pl.ds(h * _HC, _HC)], idx_s)
        pltpu.sync_copy(dst_hbm.at[wid, pl.ds(h * _HC, _HC)], idx_d)
        gather_start(0, rows0, sem0)
        if first:
            zcp.wait()
            plsc.subcore_barrier()
            first = False

        def step(i, carry):
            j = 2 * i
            gather_start(j + 1, rows1, sem1)
            gather_wait(j, rows0, sem0)
            pltpu.sync_copy(rows0, acc.at[idx_d.at[j]], add=True)

            @pl.when(i + 1 < _HC // 2)
            def _():
                gather_start(j + 2, rows0, sem0)

            gather_wait(j + 1, rows1, sem1)
            pltpu.sync_copy(rows1, acc.at[idx_d.at[j + 1]], add=True)
            return carry

        lax.fori_loop(0, _HC // 2, step, 0)
    plsc.subcore_barrier()
    pltpu.sync_copy(acc.at[pl.ds(sid * _STRIPE, _STRIPE)],
                    out_hbm.at[cid, pl.ds(sid * _STRIPE, _STRIPE)])


@functools.lru_cache(maxsize=None)
def _edge_scatter_build():
    return pl.kernel(
        _edge_scatter_body,
        out_type=jax.ShapeDtypeStruct((_NC, _NPAD, _D), jnp.float32),
        mesh=_sc_mesh(),
        scratch_types=[
            pltpu.VMEM((_HC, _CH), jnp.int32),
            pltpu.VMEM((_HC, _CH), jnp.int32),
            pltpu.VMEM((_CH, _D), jnp.float32),
            pltpu.VMEM((_CH, _D), jnp.float32),
            pltpu.VMEM_SHARED((_NPAD, _D), jnp.float32),
            pltpu.SemaphoreType.DMA,
            pltpu.SemaphoreType.DMA,
            pltpu.SemaphoreType.DMA,
        ],
    )


def _dinv_col(degp):
    deg = degp[0, :] + degp[1, :] + 1.0
    return lax.rsqrt(deg).reshape(_NPAD, 1)


def _sigmoid(v):
    return 1.0 / (1.0 + jnp.exp(-v))


def _tc_a_body(x_ref, w_ref, degp_ref, xw_ref, xs_ref):
    dc = _dinv_col(degp_ref[...])
    xw = jnp.dot(x_ref[...], w_ref[...], preferred_element_type=jnp.float32)
    xw_ref[...] = xw
    xs_ref[...] = xw * dc


def _tc_b_body(acc_ref, xw1_ref, w2_ref, b1_ref, degp_ref, xw2_ref, xs2_ref):
    dc = _dinv_col(degp_ref[...])
    agg = acc_ref[0] + acc_ref[1]
    h1 = jnp.maximum(dc * agg + dc * dc * xw1_ref[...] + b1_ref[...][None, :],
                     0.0)
    xw2 = jnp.dot(h1, w2_ref[...], preferred_element_type=jnp.float32)
    xw2_ref[...] = xw2
    xs2_ref[...] = xw2 * dc


def _tc_c_body(acc_ref, xw2_ref, b2_ref, degp_ref, batch_ref,
               wi_ref, wh_ref, bi_ref, bh_ref, out_ref):
    f32 = jnp.float32
    dc = _dinv_col(degp_ref[...])
    h2 = (dc * (acc_ref[0] + acc_ref[1]) + dc * dc * xw2_ref[...]
          + b2_ref[...][None, :])
    bat = batch_ref[...]
    gids = lax.broadcasted_iota(jnp.int32, (_G, _NPAD), 0)
    seg = gids == bat[None, :]
    segf = seg.astype(f32)

    h = jnp.zeros((_G, _D), f32)
    c = jnp.zeros((_G, _D), f32)
    qs = jnp.zeros((_G, 2 * _D), f32)
    for _ in range(_STEPS):
        gates = (jnp.dot(qs, wi_ref[...], preferred_element_type=f32)
                 + jnp.dot(h, wh_ref[...], preferred_element_type=f32)
                 + bi_ref[...][None, :] + bh_ref[...][None, :])
        ii = _sigmoid(gates[:, 0:_D])
        ff = _sigmoid(gates[:, _D:2 * _D])
        gg = jnp.tanh(gates[:, 2 * _D:3 * _D])
        oo = _sigmoid(gates[:, 3 * _D:4 * _D])
        c = ff * c + ii * gg
        h = oo * jnp.tanh(c)
        q = h
        qh = lax.dot_general(q, h2, (((1,), (1,)), ((), ())),
                             preferred_element_type=f32)
        e = jnp.sum(jnp.where(seg, qh, 0.0), axis=0)
        m = jnp.max(jnp.where(seg, e[None, :], -jnp.inf), axis=1)
        m = jnp.where(jnp.abs(m) < jnp.inf, m, 0.0)
        mrow = jnp.sum(segf * m[:, None], axis=0)
        ex = jnp.exp(e - mrow)
        ext = segf * ex[None, :]
        ssum = jnp.sum(ext, axis=1)
        rnum = jnp.dot(ext, h2, preferred_element_type=f32)
        r = rnum / (ssum[:, None] + 1e-16)
        qs = jnp.concatenate([q, r], axis=1)
    out_ref[...] = qs


_tc_a = pl.pallas_call(
    _tc_a_body,
    out_shape=[jax.ShapeDtypeStruct((_NPAD, _D), jnp.float32)] * 2,
)

_tc_b = pl.pallas_call(
    _tc_b_body,
    out_shape=[jax.ShapeDtypeStruct((_NPAD, _D), jnp.float32)] * 2,
)

_tc_c = pl.pallas_call(
    _tc_c_body,
    out_shape=jax.ShapeDtypeStruct((_G, 2 * _D), jnp.float32),
)


def kernel(x, edge_index, batch, W1, b1, W2, b2, Wi, Wh, bi, bh):
    f32 = jnp.float32
    src = edge_index[0]
    dst = edge_index[1]
    pad_e = _EPAD - _E
    fill = _N + (jnp.arange(pad_e, dtype=jnp.int32) % (_NPAD - _N))
    srcp = jnp.concatenate([src, fill]).reshape(_NW, _NCHUNK, _CH)
    dstp = jnp.concatenate([dst, fill]).reshape(_NW, _NCHUNK, _CH)
    xp = jnp.concatenate([x, jnp.zeros((_NPAD - _N, _D), f32)], axis=0)
    batp = jnp.concatenate([batch, jnp.full((_NPAD - _N,), _G, jnp.int32)])
    zrow = jnp.zeros((_NPAD, _D), f32)
    zvec = jnp.zeros((_NPAD,), f32)

    degp = _deg_kernel_build()(dstp, zvec)
    xw1, xs1 = _tc_a(xp, W1, degp)
    acc1 = _edge_scatter_build()(xs1, srcp, dstp, zrow)
    xw2, xs2 = _tc_b(acc1, xw1, W2, b1, degp)
    acc2 = _edge_scatter_build()(xs2, srcp, dstp, zrow)
    return _tc_c(acc2, xw2, b2, degp, batp, Wi, Wh, bi, bh)


# speedup vs baseline: 1.2743x; 1.0042x over previous
import jax
import jax.numpy as jnp
from jax import lax
from jax.experimental import pallas as pl
from jax.experimental.pallas import tpu as pltpu
from jax.experimental.pallas import tpu_sc as plsc

_N = 10000
_E = 320000
_D = 128
_G = 64
_STEPS = 3

_NC = 2
_NS = 16
_NW = _NC * _NS
_CH = 128
_NCHUNK = 80
_HC = _NCHUNK // 2
_EPAD = _NW * _NCHUNK * _CH
_NPAD = 10240
_STRIPE = _NPAD // _NS


def _deg_body(dst_hbm, zerov_hbm, out_hbm, idx_d, ones, accd):
    cid = lax.axis_index("c")
    sid = lax.axis_index("s")
    wid = cid * _NS + sid
    pltpu.sync_copy(dst_hbm.at[wid], idx_d)
    pltpu.sync_copy(zerov_hbm.at[pl.ds(sid * _STRIPE, _STRIPE)],
                    accd.at[pl.ds(sid * _STRIPE, _STRIPE)])
    for k in range(_CH // 16):
        ones[pl.ds(k * 16, 16)] = jnp.ones((16,), jnp.float32)
    plsc.subcore_barrier()

    def step(j, carry):
        pltpu.sync_copy(ones, accd.at[idx_d.at[j]], add=True)
        return carry

    lax.fori_loop(0, _NCHUNK, step, 0)
    plsc.subcore_barrier()
    pltpu.sync_copy(accd.at[pl.ds(sid * _STRIPE, _STRIPE)],
                    out_hbm.at[cid, pl.ds(sid * _STRIPE, _STRIPE)])


import functools


@functools.lru_cache(maxsize=None)
def _sc_mesh():
    return plsc.VectorSubcoreMesh(core_axis_name="c", subcore_axis_name="s",
                                  num_cores=_NC, num_subcores=_NS)


@functools.lru_cache(maxsize=None)
def _deg_kernel_build():
    return pl.kernel(
        _deg_body,
        out_type=jax.ShapeDtypeStruct((_NC, _NPAD), jnp.float32),
        mesh=_sc_mesh(),
        scratch_types=[
            pltpu.VMEM((_NCHUNK, _CH), jnp.int32),
            pltpu.VMEM((_CH,), jnp.float32),
            pltpu.VMEM_SHARED((_NPAD,), jnp.float32),
        ],
    )


def _edge_scatter_body(xs_hbm, src_hbm, dst_hbm, zrow_hbm, out_hbm,
                       idx_s, idx_d, rows0, rows1, acc, sem0, sem1, sem2):
    cid = lax.axis_index("c")
    sid = lax.axis_index("s")
    wid = cid * _NS + sid
    zcp = pltpu.async_copy(zrow_hbm.at[pl.ds(sid * _STRIPE, _STRIPE)],
                           acc.at[pl.ds(sid * _STRIPE, _STRIPE)], sem2)

    def gather_start(j, buf, sem):
        pltpu.async_copy(xs_hbm.at[idx_s.at[j]], buf, sem)

    def gather_wait(j, buf, sem):
        pltpu.make_async_copy(xs_hbm.at[idx_s.at[j]], buf, sem).wait()

    first = True
    for h in range(_NCHUNK // _HC):
        pltpu.sync_copy(src_hbm.at[wid, pl.ds(h * _HC, _HC)], idx_s)
        pltpu.sync_copy(dst_hbm.at[wid, pl.ds(h * _HC, _HC)], idx_d)
        gather_start(0, rows0, sem0)
        if first:
            zcp.wait()
            plsc.subcore_barrier()
            first = False

        def step(i, carry):
            j = 2 * i
            gather_start(j + 1, rows1, sem1)
            gather_wait(j, rows0, sem0)
            pltpu.sync_copy(rows0, acc.at[idx_d.at[j]], add=True)

            @pl.when(i + 1 < _HC // 2)
            def _():
                gather_start(j + 2, rows0, sem0)

            gather_wait(j + 1, rows1, sem1)
            pltpu.sync_copy(rows1, acc.at[idx_d.at[j + 1]], add=True)
            return carry

        lax.fori_loop(0, _HC // 2, step, 0)
    plsc.subcore_barrier()
    pltpu.sync_copy(acc.at[pl.ds(sid * _STRIPE, _STRIPE)],
                    out_hbm.at[cid, pl.ds(sid * _STRIPE, _STRIPE)])


@functools.lru_cache(maxsize=None)
def _edge_scatter_build():
    return pl.kernel(
        _edge_scatter_body,
        out_type=jax.ShapeDtypeStruct((_NC, _NPAD, _D), jnp.float32),
        mesh=_sc_mesh(),
        scratch_types=[
            pltpu.VMEM((_HC, _CH), jnp.int32),
            pltpu.VMEM((_HC, _CH), jnp.int32),
            pltpu.VMEM((_CH, _D), jnp.float32),
            pltpu.VMEM((_CH, _D), jnp.float32),
            pltpu.VMEM_SHARED((_NPAD, _D), jnp.float32),
            pltpu.SemaphoreType.DMA,
            pltpu.SemaphoreType.DMA,
            pltpu.SemaphoreType.DMA,
        ],
    )


def _dinv_col(degp):
    deg = degp[0, :] + degp[1, :] + 1.0
    return lax.rsqrt(deg).reshape(_NPAD, 1)


def _sigmoid(v):
    return 1.0 / (1.0 + jnp.exp(-v))


def _tc_a1_body(x_ref, w_ref, xw_ref):
    xw = jnp.dot(x_ref[...], w_ref[...], preferred_element_type=jnp.float32)
    xw_ref[0:_N, :] = xw
    xw_ref[_N:_NPAD, :] = jnp.zeros((_NPAD - _N, _D), jnp.float32)


def _tc_a2_body(xw_ref, degp_ref, xs_ref):
    dc = _dinv_col(degp_ref[...])
    xs_ref[...] = xw_ref[...] * dc


def _tc_b_body(acc_ref, xw1_ref, w2_ref, b1_ref, degp_ref, xw2_ref, xs2_ref):
    dc = _dinv_col(degp_ref[...])
    agg = acc_ref[0] + acc_ref[1]
    h1 = jnp.maximum(dc * agg + dc * dc * xw1_ref[...] + b1_ref[...][None, :],
                     0.0)
    xw2 = jnp.dot(h1, w2_ref[...], preferred_element_type=jnp.float32)
    xw2_ref[...] = xw2
    xs2_ref[...] = xw2 * dc


def _tc_c_body(acc_ref, xw2_ref, b2_ref, degp_ref, batch_ref,
               wi_ref, wh_ref, bi_ref, bh_ref, out_ref):
    f32 = jnp.float32
    dc = _dinv_col(degp_ref[...])
    h2 = (dc * (acc_ref[0] + acc_ref[1]) + dc * dc * xw2_ref[...]
          + b2_ref[...][None, :])
    bat = batch_ref[...]
    gids = lax.broadcasted_iota(jnp.int32, (_G, _NPAD), 0)
    seg = gids == bat[None, :]
    segf = seg.astype(f32)

    h = jnp.zeros((_G, _D), f32)
    c = jnp.zeros((_G, _D), f32)
    qs = jnp.zeros((_G, 2 * _D), f32)
    for _ in range(_STEPS):
        gates = (jnp.dot(qs, wi_ref[...], preferred_element_type=f32)
                 + jnp.dot(h, wh_ref[...], preferred_element_type=f32)
                 + bi_ref[...][None, :] + bh_ref[...][None, :])
        ii = _sigmoid(gates[:, 0:_D])
        ff = _sigmoid(gates[:, _D:2 * _D])
        gg = jnp.tanh(gates[:, 2 * _D:3 * _D])
        oo = _sigmoid(gates[:, 3 * _D:4 * _D])
        c = ff * c + ii * gg
        h = oo * jnp.tanh(c)
        q = h
        qh = lax.dot_general(q, h2, (((1,), (1,)), ((), ())),
                             preferred_element_type=f32)
        e = jnp.sum(jnp.where(seg, qh, 0.0), axis=0)
        m = jnp.max(jnp.where(seg, e[None, :], -jnp.inf), axis=1)
        m = jnp.where(jnp.abs(m) < jnp.inf, m, 0.0)
        mrow = jnp.sum(segf * m[:, None], axis=0)
        ex = jnp.exp(e - mrow)
        ext = segf * ex[None, :]
        ssum = jnp.sum(ext, axis=1)
        rnum = jnp.dot(ext, h2, preferred_element_type=f32)
        r = rnum / (ssum[:, None] + 1e-16)
        qs = jnp.concatenate([q, r], axis=1)
    out_ref[...] = qs


_tc_a1 = pl.pallas_call(
    _tc_a1_body,
    out_shape=jax.ShapeDtypeStruct((_NPAD, _D), jnp.float32),
)

_tc_a2 = pl.pallas_call(
    _tc_a2_body,
    out_shape=jax.ShapeDtypeStruct((_NPAD, _D), jnp.float32),
)

_tc_b = pl.pallas_call(
    _tc_b_body,
    out_shape=[jax.ShapeDtypeStruct((_NPAD, _D), jnp.float32)] * 2,
)

_tc_c = pl.pallas_call(
    _tc_c_body,
    out_shape=jax.ShapeDtypeStruct((_G, 2 * _D), jnp.float32),
)


def kernel(x, edge_index, batch, W1, b1, W2, b2, Wi, Wh, bi, bh):
    f32 = jnp.float32
    src = edge_index[0]
    dst = edge_index[1]
    pad_e = _EPAD - _E
    fill = _N + (jnp.arange(pad_e, dtype=jnp.int32) % (_NPAD - _N))
    srcp = jnp.concatenate([src, fill]).reshape(_NW, _NCHUNK, _CH)
    dstp = jnp.concatenate([dst, fill]).reshape(_NW, _NCHUNK, _CH)
    batp = jnp.concatenate([batch, jnp.full((_NPAD - _N,), _G, jnp.int32)])
    zrow = jnp.zeros((_NPAD, _D), f32)
    zvec = jnp.zeros((_NPAD,), f32)

    xw1 = _tc_a1(x, W1)
    degp = _deg_kernel_build()(dstp, zvec)
    xs1 = _tc_a2(xw1, degp)
    acc1 = _edge_scatter_build()(xs1, srcp, dstp, zrow)
    xw2, xs2 = _tc_b(acc1, xw1, W2, b1, degp)
    acc2 = _edge_scatter_build()(xs2, srcp, dstp, zrow)
    return _tc_c(acc2, xw2, b2, degp, batp, Wi, Wh, bi, bh)


# speedup vs baseline: 1.2763x; 1.0016x over previous
import functools

import jax
import jax.numpy as jnp
from jax import lax
from jax.experimental import pallas as pl
from jax.experimental.pallas import tpu as pltpu
from jax.experimental.pallas import tpu_sc as plsc

_N = 10000
_E = 320000
_D = 128
_G = 64
_STEPS = 3

_NC = 2
_NS = 16
_NW = _NC * _NS
_CH = 128
_NCHUNK = 80
_HC = _NCHUNK // 2
_EPAD = _NW * _NCHUNK * _CH
_NPAD = 10240
_STRIPE = _NPAD // _NS


def _deg_body(dst_hbm, zerov_hbm, out_hbm, idx_d, ones, accd):
    cid = lax.axis_index("c")
    sid = lax.axis_index("s")
    wid = cid * _NS + sid
    pltpu.sync_copy(dst_hbm.at[wid], idx_d)
    pltpu.sync_copy(zerov_hbm.at[pl.ds(sid * _STRIPE, _STRIPE)],
                    accd.at[pl.ds(sid * _STRIPE, _STRIPE)])
    for k in range(_CH // 16):
        ones[pl.ds(k * 16, 16)] = jnp.ones((16,), jnp.float32)
    plsc.subcore_barrier()

    def step(j, carry):
        pltpu.sync_copy(ones, accd.at[idx_d.at[j]], add=True)
        return carry

    lax.fori_loop(0, _NCHUNK, step, 0)
    plsc.subcore_barrier()
    pltpu.sync_copy(accd.at[pl.ds(sid * _STRIPE, _STRIPE)],
                    out_hbm.at[cid, pl.ds(sid * _STRIPE, _STRIPE)])


@functools.lru_cache(maxsize=None)
def _sc_mesh():
    return plsc.VectorSubcoreMesh(core_axis_name="c", subcore_axis_name="s",
                                  num_cores=_NC, num_subcores=_NS)


@functools.lru_cache(maxsize=None)
def _deg_kernel_build():
    return pl.kernel(
        _deg_body,
        out_type=jax.ShapeDtypeStruct((_NC, _NPAD), jnp.float32),
        mesh=_sc_mesh(),
        scratch_types=[
            pltpu.VMEM((_NCHUNK, _CH), jnp.int32),
            pltpu.VMEM((_CH,), jnp.float32),
            pltpu.VMEM_SHARED((_NPAD,), jnp.float32),
        ],
    )


def _edge_scatter_body(xs_hbm, src_hbm, dst_hbm, zrow_hbm, out_hbm,
                       idx_s, idx_d, rows0, rows1, acc, sem0, sem1, sem2):
    cid = lax.axis_index("c")
    sid = lax.axis_index("s")
    wid = cid * _NS + sid
    zcp = pltpu.async_copy(zrow_hbm.at[pl.ds(sid * _STRIPE, _STRIPE)],
                           acc.at[pl.ds(sid * _STRIPE, _STRIPE)], sem2)

    def gather_start(j, buf, sem):
        pltpu.async_copy(xs_hbm.at[idx_s.at[j]], buf, sem)

    def gather_wait(j, buf, sem):
        pltpu.make_async_copy(xs_hbm.at[idx_s.at[j]], buf, sem).wait()

    first = True
    for h in range(_NCHUNK // _HC):
        pltpu.sync_copy(src_hbm.at[wid, pl.ds(h * _HC, _HC)], idx_s)
        pltpu.sync_copy(dst_hbm.at[wid, pl.ds(h * _HC, _HC)], idx_d)
        gather_start(0, rows0, sem0)
        if first:
            zcp.wait()
            plsc.subcore_barrier()
            first = False

        def step(i, carry):
            j = 2 * i
            gather_start(j + 1, rows1, sem1)
            gather_wait(j, rows0, sem0)
            pltpu.sync_copy(rows0, acc.at[idx_d.at[j]], add=True)

            @pl.when(i + 1 < _HC // 2)
            def _():
                gather_start(j + 2, rows0, sem0)

            gather_wait(j + 1, rows1, sem1)
            pltpu.sync_copy(rows1, acc.at[idx_d.at[j + 1]], add=True)
            return carry

        lax.fori_loop(0, _HC // 2, step, 0)
    plsc.subcore_barrier()
    pltpu.sync_copy(acc.at[pl.ds(sid * _STRIPE, _STRIPE)],
                    out_hbm.at[cid, pl.ds(sid * _STRIPE, _STRIPE)])


@functools.lru_cache(maxsize=None)
def _edge_scatter_build():
    return pl.kernel(
        _edge_scatter_body,
        out_type=jax.ShapeDtypeStruct((_NC, _NPAD, _D), jnp.float32),
        mesh=_sc_mesh(),
        scratch_types=[
            pltpu.VMEM((_HC, _CH), jnp.int32),
            pltpu.VMEM((_HC, _CH), jnp.int32),
            pltpu.VMEM((_CH, _D), jnp.float32),
            pltpu.VMEM((_CH, _D), jnp.float32),
            pltpu.VMEM_SHARED((_NPAD, _D), jnp.float32),
            pltpu.SemaphoreType.DMA,
            pltpu.SemaphoreType.DMA,
            pltpu.SemaphoreType.DMA,
        ],
    )


def _dinv_col(degp):
    deg = degp[0, :] + degp[1, :] + 1.0
    return lax.rsqrt(deg).reshape(_NPAD, 1)


def _sigmoid(v):
    return 1.0 / (1.0 + jnp.exp(-v))


def _tc_a1_body(x_ref, w_ref, xw_ref):
    xw = jnp.dot(x_ref[...], w_ref[...], preferred_element_type=jnp.float32)
    xw_ref[0:_N, :] = xw
    xw_ref[_N:_NPAD, :] = jnp.zeros((_NPAD - _N, _D), jnp.float32)


def _tc_a2_body(xw_ref, degp_ref, xs_ref):
    dc = _dinv_col(degp_ref[...])
    xs_ref[...] = xw_ref[...] * dc


def _tc_b_body(acc_ref, xw1_ref, w2_ref, b1_ref, degp_ref, xw2_ref, xs2_ref):
    dc = _dinv_col(degp_ref[...])
    agg = acc_ref[0] + acc_ref[1]
    h1 = jnp.maximum(dc * agg + dc * dc * xw1_ref[...] + b1_ref[...][None, :],
                     0.0)
    xw2 = jnp.dot(h1, w2_ref[...], preferred_element_type=jnp.float32)
    xw2_ref[...] = xw2
    xs2_ref[...] = xw2 * dc


def _tc_c_body(acc_ref, xw2_ref, b2_ref, degp_ref, batch_ref,
               wi_ref, wh_ref, bi_ref, bh_ref, out_ref):
    f32 = jnp.float32
    dc = _dinv_col(degp_ref[...])
    h2 = (dc * (acc_ref[0] + acc_ref[1]) + dc * dc * xw2_ref[...]
          + b2_ref[...][None, :])
    bat = batch_ref[...]
    gids = lax.broadcasted_iota(jnp.int32, (_G, _NPAD), 0)
    seg = gids == bat[None, :]
    segf = seg.astype(f32)

    h = jnp.zeros((_G, _D), f32)
    c = jnp.zeros((_G, _D), f32)
    qs = jnp.zeros((_G, 2 * _D), f32)
    for _ in range(_STEPS):
        gates = (jnp.dot(qs, wi_ref[...], preferred_element_type=f32)
                 + jnp.dot(h, wh_ref[...], preferred_element_type=f32)
                 + bi_ref[...][None, :] + bh_ref[...][None, :])
        ii = _sigmoid(gates[:, 0:_D])
        ff = _sigmoid(gates[:, _D:2 * _D])
        gg = jnp.tanh(gates[:, 2 * _D:3 * _D])
        oo = _sigmoid(gates[:, 3 * _D:4 * _D])
        c = ff * c + ii * gg
        h = oo * jnp.tanh(c)
        q = h
        qh = lax.dot_general(q, h2, (((1,), (1,)), ((), ())),
                             preferred_element_type=f32)
        e = jnp.sum(jnp.where(seg, qh, 0.0), axis=0)
        m = jnp.max(jnp.where(seg, e[None, :], -jnp.inf), axis=1)
        m = jnp.where(jnp.abs(m) < jnp.inf, m, 0.0)
        mrow = jnp.sum(segf * m[:, None], axis=0)
        ex = jnp.exp(e - mrow)
        ext = segf * ex[None, :]
        ssum = jnp.sum(ext, axis=1)
        rnum = jnp.dot(ext, h2, preferred_element_type=f32)
        r = rnum / (ssum[:, None] + 1e-16)
        qs = jnp.concatenate([q, r], axis=1)
    out_ref[...] = qs


_tc_a1 = pl.pallas_call(
    _tc_a1_body,
    out_shape=jax.ShapeDtypeStruct((_NPAD, _D), jnp.float32),
)

_tc_a2 = pl.pallas_call(
    _tc_a2_body,
    out_shape=jax.ShapeDtypeStruct((_NPAD, _D), jnp.float32),
)

_tc_b = pl.pallas_call(
    _tc_b_body,
    out_shape=[jax.ShapeDtypeStruct((_NPAD, _D), jnp.float32)] * 2,
)

_tc_c = pl.pallas_call(
    _tc_c_body,
    out_shape=jax.ShapeDtypeStruct((_G, 2 * _D), jnp.float32),
)


def kernel(x, edge_index, batch, W1, b1, W2, b2, Wi, Wh, bi, bh):
    f32 = jnp.float32
    src = edge_index[0]
    dst = edge_index[1]
    pad_e = _EPAD - _E
    fill = _N + (jnp.arange(pad_e, dtype=jnp.int32) % (_NPAD - _N))
    srcp = jnp.concatenate([src, fill]).reshape(_NW, _NCHUNK, _CH)
    dstp = jnp.concatenate([dst, fill]).reshape(_NW, _NCHUNK, _CH)
    batp = jnp.concatenate([batch, jnp.full((_NPAD - _N,), _G, jnp.int32)])
    zrow = jnp.zeros((_NPAD, _D), f32)
    zvec = jnp.zeros((_NPAD,), f32)

    xw1 = _tc_a1(x, W1)
    degp = _deg_kernel_build()(dstp, zvec)
    xs1 = _tc_a2(xw1, degp)
    acc1 = _edge_scatter_build()(xs1, srcp, dstp, zrow)
    xw2, xs2 = _tc_b(acc1, xw1, W2, b1, degp)
    acc2 = _edge_scatter_build()(xs2, srcp, dstp, zrow)
    return _tc_c(acc2, xw2, b2, degp, batp, Wi, Wh, bi, bh)
